# baseline (speedup 1.0000x reference)
import functools

import jax
import jax.numpy as jnp
from jax import lax
from jax.experimental import pallas as pl
from jax.experimental.pallas import tpu as pltpu
from jax.experimental.pallas import tpu_sc as plsc

BLOCK = 64
NSEL = 16
LANES = 16


def kernel(keys, compression_scores):
  batch, seq_len, key_dim = keys.shape
  num_blocks = seq_len // BLOCK
  nchunks = num_blocks // LANES
  out_rows = NSEL * BLOCK
  nring = 12
  lag = 4

  info = plsc.get_sparse_core_info()
  nc, ns = info.num_cores, info.num_subcores
  assert nc * ns == batch, (nc, ns, batch)

  table = keys.reshape(batch * seq_len, key_dim)

  mesh = plsc.VectorSubcoreMesh(core_axis_name="c", subcore_axis_name="s")

  @functools.partial(
      pl.kernel,
      out_type=jax.ShapeDtypeStruct((batch * out_rows, key_dim), jnp.float32),
      mesh=mesh,
      scratch_types=[
          pltpu.VMEM((num_blocks,), jnp.float32),
          pltpu.VMEM_SHARED((ns, NSEL // 2 * BLOCK, key_dim), jnp.float32),
          pltpu.SemaphoreType.DMA,
          pltpu.SemaphoreType.DMA,
      ],
  )
  def run(table_hbm, scores_hbm, out_hbm, scores_v, sbuf, gsem, osem):
    sid = lax.axis_index("s")
    b = sid * nc + lax.axis_index("c")
    pltpu.sync_copy(scores_hbm.at[b], scores_v)

    chunks = [scores_v[pl.ds(LANES * i, LANES)] for i in range(nchunks)]
    gidx = [lax.iota(jnp.int32, LANES) + LANES * i for i in range(nchunks)]
    valid = [jnp.ones((LANES,), jnp.bool_) for _ in range(nchunks)]

    neg_inf = jnp.float32(-jnp.inf)
    big = jnp.int32(num_blocks)
    lane = lax.iota(jnp.int32, LANES)
    perms = [lane ^ s for s in (8, 4, 2, 1)]

    def butterfly(v, op):
      for s in range(4):
        v = op(v, v.at[perms[s]].get(mode="promise_in_bounds"))
      return v

    seq_base = b * (num_blocks * BLOCK)
    out_base = b * out_rows
    half = NSEL // 2
    gathers = [None] * NSEL
    out_prev = None

    for j in range(NSEL):
      masked = [jnp.where(valid[i], chunks[i], neg_inf) for i in range(nchunks)]
      mv = masked[0]
      for i in range(1, nchunks):
        mv = jnp.maximum(mv, masked[i])
      m = butterfly(mv, jnp.maximum)
      iv = jnp.where(valid[0] & (chunks[0] == m), gidx[0], big)
      for i in range(1, nchunks):
        iv = jnp.minimum(iv, jnp.where(valid[i] & (chunks[i] == m), gidx[i],
                                       big))
      sel_v = butterfly(iv, jnp.minimum)
      valid = [valid[i] & (gidx[i] != sel_v) for i in range(nchunks)]
      sel = sel_v[0]
      if j == half:
        for jj in range(half):
          gathers[jj].wait()
        out_prev = pltpu.async_copy(
            sbuf.at[sid],
            out_hbm.at[pl.ds(out_base, half * BLOCK)], osem)
        out_prev.wait()
      gathers[j] = pltpu.async_copy(
          table_hbm.at[pl.ds(seq_base + sel * BLOCK, BLOCK)],
          sbuf.at[sid, pl.ds((j % half) * BLOCK, BLOCK)], gsem)
    for jj in range(half, NSEL):
      gathers[jj].wait()
    pltpu.async_copy(
        sbuf.at[sid],
        out_hbm.at[pl.ds(out_base + half * BLOCK, half * BLOCK)], osem).wait()

  out = run(table, compression_scores)
  return out.reshape(batch, out_rows, key_dim)


# speedup vs baseline: 1.0730x; 1.0730x over previous
import functools

import jax
import jax.numpy as jnp
from jax import lax
from jax.experimental import pallas as pl
from jax.experimental.pallas import tpu as pltpu
from jax.experimental.pallas import tpu_sc as plsc

BLOCK = 64
NSEL = 16
LANES = 16


def kernel(keys, compression_scores):
  batch, seq_len, key_dim = keys.shape
  num_blocks = seq_len // BLOCK
  nchunks = num_blocks // LANES
  out_rows = NSEL * BLOCK
  nring = 12
  lag = 4

  info = plsc.get_sparse_core_info()
  nc, ns = info.num_cores, info.num_subcores
  assert nc * ns == batch, (nc, ns, batch)

  table = keys.reshape(batch * seq_len, key_dim)

  mesh = plsc.VectorSubcoreMesh(core_axis_name="c", subcore_axis_name="s")

  @functools.partial(
      pl.kernel,
      out_type=jax.ShapeDtypeStruct((batch * out_rows, key_dim), jnp.float32),
      mesh=mesh,
      scratch_types=[
          pltpu.VMEM((num_blocks,), jnp.float32),
          pltpu.VMEM_SHARED((ns, 2, NSEL // 4 * BLOCK, key_dim), jnp.float32),
          pltpu.SemaphoreType.DMA,
          pltpu.SemaphoreType.DMA,
      ],
  )
  def run(table_hbm, scores_hbm, out_hbm, scores_v, sbuf, gsem, osem):
    sid = lax.axis_index("s")
    b = sid * nc + lax.axis_index("c")
    pltpu.sync_copy(scores_hbm.at[b], scores_v)

    chunks = [scores_v[pl.ds(LANES * i, LANES)] for i in range(nchunks)]
    gidx = [lax.iota(jnp.int32, LANES) + LANES * i for i in range(nchunks)]
    valid = [jnp.ones((LANES,), jnp.bool_) for _ in range(nchunks)]

    neg_inf = jnp.float32(-jnp.inf)
    big = jnp.int32(num_blocks)
    lane = lax.iota(jnp.int32, LANES)
    perms = [lane ^ s for s in (8, 4, 2, 1)]

    def butterfly(v, op):
      for s in range(4):
        v = op(v, v.at[perms[s]].get(mode="promise_in_bounds"))
      return v

    seq_base = b * (num_blocks * BLOCK)
    out_base = b * out_rows
    quarter = NSEL // 4
    nrounds = 4
    gathers = [None] * NSEL
    outs = [None] * nrounds

    def start_out(r):
      for jj in range(r * quarter, (r + 1) * quarter):
        gathers[jj].wait()
      outs[r] = pltpu.async_copy(
          sbuf.at[sid, r % 2],
          out_hbm.at[pl.ds(out_base + r * quarter * BLOCK, quarter * BLOCK)],
          osem)

    for j in range(NSEL):
      masked = [jnp.where(valid[i], chunks[i], neg_inf) for i in range(nchunks)]
      mv = masked[0]
      for i in range(1, nchunks):
        mv = jnp.maximum(mv, masked[i])
      m = butterfly(mv, jnp.maximum)
      iv = jnp.where(valid[0] & (chunks[0] == m), gidx[0], big)
      for i in range(1, nchunks):
        iv = jnp.minimum(iv, jnp.where(valid[i] & (chunks[i] == m), gidx[i],
                                       big))
      sel_v = butterfly(iv, jnp.minimum)
      valid = [valid[i] & (gidx[i] != sel_v) for i in range(nchunks)]
      sel = sel_v[0]
      r, rj = j // quarter, j % quarter
      if rj == 0 and r >= 1:
        start_out(r - 1)
      if rj == 0 and r >= 2:
        outs[r - 2].wait()
      gathers[j] = pltpu.async_copy(
          table_hbm.at[pl.ds(seq_base + sel * BLOCK, BLOCK)],
          sbuf.at[sid, r % 2, pl.ds(rj * BLOCK, BLOCK)], gsem)
    start_out(nrounds - 1)
    outs[nrounds - 2].wait()
    outs[nrounds - 1].wait()

  out = run(table, compression_scores)
  return out.reshape(batch, out_rows, key_dim)


# speedup vs baseline: 1.0887x; 1.0146x over previous
import functools

import jax
import jax.numpy as jnp
from jax import lax
from jax.experimental import pallas as pl
from jax.experimental.pallas import tpu as pltpu
from jax.experimental.pallas import tpu_sc as plsc

BLOCK = 64
NSEL = 16
LANES = 16


def kernel(keys, compression_scores):
  batch, seq_len, key_dim = keys.shape
  num_blocks = seq_len // BLOCK
  nchunks = num_blocks // LANES
  out_rows = NSEL * BLOCK
  nring = 8
  lag = 3

  info = plsc.get_sparse_core_info()
  nc, ns = info.num_cores, info.num_subcores
  assert nc * ns == batch, (nc, ns, batch)

  table = keys.reshape(batch * seq_len, key_dim)

  mesh = plsc.VectorSubcoreMesh(core_axis_name="c", subcore_axis_name="s")

  @functools.partial(
      pl.kernel,
      out_type=jax.ShapeDtypeStruct((batch * out_rows, key_dim), jnp.float32),
      mesh=mesh,
      scratch_types=[
          pltpu.VMEM((num_blocks,), jnp.float32),
          pltpu.VMEM((nring, BLOCK, key_dim), jnp.float32),
          pltpu.SemaphoreType.DMA,
          pltpu.SemaphoreType.DMA,
      ],
  )
  def run(table_hbm, scores_hbm, out_hbm, scores_v, buf, gsem, osem):
    b = lax.axis_index("s") * nc + lax.axis_index("c")
    pltpu.sync_copy(scores_hbm.at[b], scores_v)

    chunks = [scores_v[pl.ds(LANES * i, LANES)] for i in range(nchunks)]
    gidx = [lax.iota(jnp.int32, LANES) + LANES * i for i in range(nchunks)]
    valid = [jnp.ones((LANES,), jnp.bool_) for _ in range(nchunks)]

    neg_inf = jnp.float32(-jnp.inf)
    big = jnp.int32(num_blocks)
    lane = lax.iota(jnp.int32, LANES)
    perms = [lane ^ s for s in (8, 4, 2, 1)]

    def butterfly(v, op):
      for s in range(4):
        v = op(v, v.at[perms[s]].get(mode="promise_in_bounds"))
      return v

    seq_base = b * (num_blocks * BLOCK)
    out_base = b * out_rows
    gathers = [None] * NSEL
    outs = [None] * NSEL

    def start_out(j):
      gathers[j].wait()
      outs[j] = pltpu.async_copy(
          buf.at[j % nring],
          out_hbm.at[pl.ds(out_base + j * BLOCK, BLOCK)], osem)

    for j in range(NSEL):
      masked = [jnp.where(valid[i], chunks[i], neg_inf) for i in range(nchunks)]
      mv = masked[0]
      for i in range(1, nchunks):
        mv = jnp.maximum(mv, masked[i])
      m = butterfly(mv, jnp.maximum)
      iv = jnp.where(valid[0] & (chunks[0] == m), gidx[0], big)
      for i in range(1, nchunks):
        iv = jnp.minimum(iv, jnp.where(valid[i] & (chunks[i] == m), gidx[i],
                                       big))
      sel_v = butterfly(iv, jnp.minimum)
      valid = [valid[i] & (gidx[i] != sel_v) for i in range(nchunks)]
      sel = sel_v[0]
      if j >= nring:
        outs[j - nring].wait()
      gathers[j] = pltpu.async_copy(
          table_hbm.at[pl.ds(seq_base + sel * BLOCK, BLOCK)],
          buf.at[j % nring], gsem)
      if j >= lag:
        start_out(j - lag)
    for j in range(NSEL - lag, NSEL):
      start_out(j)
    for j in range(NSEL - nring, NSEL):
      outs[j].wait()

  out = run(table, compression_scores)
  return out.reshape(batch, out_rows, key_dim)
